# Initial kernel scaffold; baseline (speedup 1.0000x reference)
import jax
import jax.numpy as jnp
from jax.experimental import pallas as pl


def kernel(embeddings, unknown_mask):
    raise NotImplementedError("write your pallas kernel here")


# speedup vs baseline: 1.4523x; 1.4523x over previous
import functools

import numpy as np
import jax
import jax.numpy as jnp
from jax import lax
from jax.experimental import pallas as pl
from jax.experimental.pallas import tpu as pltpu
from jax.experimental.pallas import tpu_sc as plsc

TOTAL = 16384
DIM = 1024
NSAMP = 512
SMALL_CNT = 1625
WEIGHT = 1.0


def _tf2x32(k0, k1, x0, x1):
    k0, k1 = np.uint32(k0), np.uint32(k1)
    x0, x1 = np.asarray(x0, np.uint32), np.asarray(x1, np.uint32)
    rotations = ((13, 15, 26, 6), (17, 29, 16, 24))
    ks = (k0, k1, np.uint32(k0 ^ k1 ^ np.uint32(0x1BD11BDA)))
    x0 = x0 + ks[0]
    x1 = x1 + ks[1]
    for i in range(5):
        for r in rotations[i % 2]:
            x0 = x0 + x1
            x1 = ((x1 << np.uint32(r)) | (x1 >> np.uint32(32 - r))) ^ x0
        x0 = x0 + ks[(i + 1) % 3]
        x1 = x1 + ks[(i + 2) % 3] + np.uint32(i + 1)
    return x0, x1


def _np_split(k):
    b1, b2 = _tf2x32(k[0], k[1], np.zeros(2, np.uint32),
                     np.arange(2, dtype=np.uint32))
    return (b1[0], b2[0]), (b1[1], b2[1])


def _np_bits(k, n):
    b1, b2 = _tf2x32(k[0], k[1], np.zeros(n, np.uint32),
                     np.arange(n, dtype=np.uint32))
    return b1 ^ b2


def _shuffle_orders():
    old = np.seterr(over="ignore")
    try:
        ka, kb = _np_split((np.uint32(0), np.uint32(1234)))
        orders = []
        for base in (ka, kb):
            knext, sub1 = _np_split(base)
            _, sub2 = _np_split(knext)
            for sub in (sub1, sub2):
                bits = _np_bits(sub, TOTAL)
                orders.append(np.argsort(bits, kind="stable").astype(np.int32))
        return orders
    finally:
        np.seterr(**old)


_ORD1A, _ORD2A, _ORD1B, _ORD2B = _shuffle_orders()


_LANE = 16
_NCHUNK = TOTAL // _LANE
_PAIR_PER_TILE = NSAMP // 16


def _lane_sum(v):
    for s in range(4):
        idx = lax.iota(jnp.int32, _LANE) ^ (1 << s)
        v = v + jnp.take_along_axis(v, idx, axis=0, mode="promise_in_bounds")
    return v


def _rsqrt_vec(x):
    i = lax.bitcast_convert_type(x, jnp.int32)
    i = jnp.int32(0x5F3759DF) - lax.shift_right_arithmetic(i, 1)
    y = lax.bitcast_convert_type(i, jnp.float32)
    for _ in range(3):
        y = y * (jnp.float32(1.5) - jnp.float32(0.5) * x * y * y)
    return y


def _sc_call(emb, mask_i32, o1a, o2a, o1b, o2b):
    mesh = plsc.VectorSubcoreMesh(core_axis_name="c", subcore_axis_name="s")

    @functools.partial(
        pl.kernel,
        out_type=[
            jax.ShapeDtypeStruct((_LANE, _LANE), jnp.float32),
            jax.ShapeDtypeStruct((_LANE,), jnp.int32),
        ],
        mesh=mesh,
        compiler_params=pltpu.CompilerParams(needs_layout_passes=False),
        scratch_types=[
            pltpu.VMEM((TOTAL,), jnp.int32),
            pltpu.VMEM((TOTAL,), jnp.int32),
            pltpu.VMEM((TOTAL + _LANE,), jnp.int32),
            pltpu.VMEM((NSAMP,), jnp.int32),
            pltpu.VMEM((NSAMP,), jnp.int32),
            pltpu.VMEM((NSAMP,), jnp.int32),
            pltpu.VMEM((NSAMP,), jnp.int32),
            pltpu.VMEM((2 * _PAIR_PER_TILE,), jnp.int32),
            pltpu.VMEM((2 * _PAIR_PER_TILE, DIM), jnp.float32),
            pltpu.VMEM((_LANE,), jnp.float32),
            pltpu.VMEM((_LANE,), jnp.int32),
            pltpu.VMEM_SHARED((2, NSAMP), jnp.int32),
            pltpu.VMEM_SHARED((2, NSAMP), jnp.int32),
            pltpu.VMEM_SHARED((2, NSAMP), jnp.int32),
            pltpu.SemaphoreType.DMA,
        ],
    )
    def k(emb_hbm, mask_hbm, o1a_hbm, o2a_hbm, o1b_hbm, o2b_hbm,
          parts_hbm, meta_hbm,
          mask_v, in_v, comp_v, jl_v, ix_v, ix2_v, sel_v, idx_v,
          rows_v, scal_v, meta_v,
          sh_j, sh_idx, sh_sel, sema):
        cid = lax.axis_index("c")
        sid = lax.axis_index("s")
        on0 = cid == 0

        pltpu.sync_copy(mask_hbm, mask_v)
        pl.delay(500)

        def cnt_body(kk, acc):
            return acc + mask_v[pl.ds(kk * _LANE, _LANE)]

        cnt_acc = lax.fori_loop(0, _NCHUNK, cnt_body,
                                jnp.zeros((_LANE,), jnp.int32))
        cnt = _lane_sum(cnt_acc)[0]
        limit = jnp.minimum(cnt, jnp.int32(NSAMP))
        two_round = cnt > jnp.int32(SMALL_CNT)

        def compact_step(v, m, off):
            key = (lax.iota(jnp.int32, _LANE)
                   + jnp.where(m, 0, _LANE)).astype(jnp.uint32)
            _, sv = plsc.sort_key_val(key, v)
            comp_v[pl.ds(off, _LANE)] = sv
            pc = plsc.all_reduce_population_count(m)
            return off + pc[0]

        def compact_ord(src_hbm):
            pltpu.sync_copy(src_hbm, in_v)
            pl.delay(500)

            def body(kk, off):
                v = in_v[pl.ds(kk * _LANE, _LANE)]
                return compact_step(v, v < cnt, off)

            lax.fori_loop(0, _NCHUNK, body, jnp.int32(0))

        @pl.when(jnp.logical_and(on0, sid == 1))
        def _():
            compact_ord(o1a_hbm)

        @pl.when(jnp.logical_and(on0, sid == 2))
        def _():
            compact_ord(o1b_hbm)

        @pl.when(jnp.logical_and(on0, sid == 3))
        def _():
            compact_ord(o2a_hbm)
            pltpu.sync_copy(comp_v.at[pl.ds(0, NSAMP)], sh_j.at[0])

        @pl.when(jnp.logical_and(on0, sid == 4))
        def _():
            compact_ord(o2b_hbm)
            pltpu.sync_copy(comp_v.at[pl.ds(0, NSAMP)], sh_j.at[1])

        @pl.when(jnp.logical_and(on0, sid == 5))
        def _():
            def body(kk, off):
                mv = mask_v[pl.ds(kk * _LANE, _LANE)]
                ids = kk * _LANE + lax.iota(jnp.int32, _LANE)
                return compact_step(ids, mv > 0, off)

            lax.fori_loop(0, _NCHUNK, body, jnp.int32(0))

        plsc.subcore_barrier()

        def compose(row):
            pltpu.sync_copy(sh_j.at[row], jl_v)
            pl.delay(1500)

            def body(q, _):
                ivec = q * _LANE + lax.iota(jnp.int32, _LANE)
                jv = jnp.where(two_round,
                               jl_v[pl.ds(q * _LANE, _LANE)], ivec)
                av = plsc.load_gather(comp_v, [jv])
                av = jnp.where(ivec < limit, av, 0)
                ix_v[pl.ds(q * _LANE, _LANE)] = av
                return 0

            lax.fori_loop(0, NSAMP // _LANE, body, 0)
            pltpu.sync_copy(ix_v, sh_idx.at[row])

        @pl.when(jnp.logical_and(on0, sid == 1))
        def _():
            compose(0)

        @pl.when(jnp.logical_and(on0, sid == 2))
        def _():
            compose(1)

        plsc.subcore_barrier()

        @pl.when(jnp.logical_and(on0, sid == 5))
        def _():
            pltpu.sync_copy(sh_idx.at[0], ix_v)
            pltpu.sync_copy(sh_idx.at[1], ix2_v)
            pl.delay(1500)

            def sel_one(src_v, row):
                def body(q, _):
                    ivec = q * _LANE + lax.iota(jnp.int32, _LANE)
                    xs = src_v[pl.ds(q * _LANE, _LANE)]
                    sv = plsc.load_gather(comp_v, [xs])
                    sv = jnp.where(ivec < limit, sv, 0)
                    sel_v[pl.ds(q * _LANE, _LANE)] = sv
                    return 0

                lax.fori_loop(0, NSAMP // _LANE, body, 0)
                pltpu.sync_copy(sel_v, sh_sel.at[row])

            sel_one(ix_v, 0)
            sel_one(ix2_v, 1)

        plsc.subcore_barrier()

        @pl.when(on0)
        def _():
            base = sid * _PAIR_PER_TILE
            pltpu.sync_copy(sh_sel.at[0, pl.ds(base, _PAIR_PER_TILE)],
                            idx_v.at[pl.ds(0, _PAIR_PER_TILE)])
            pltpu.sync_copy(sh_sel.at[1, pl.ds(base, _PAIR_PER_TILE)],
                            idx_v.at[pl.ds(_PAIR_PER_TILE, _PAIR_PER_TILE)])
            pltpu.async_copy(emb_hbm.at[idx_v], rows_v, sema).wait()
            pl.delay(2500)

            ploss = jnp.zeros((_LANE,), jnp.float32)
            for p in range(_PAIR_PER_TILE):
                def dot_body(q, accs):
                    ab, aa, bb = accs
                    va = rows_v[p, pl.ds(q * _LANE, _LANE)]
                    vb = rows_v[_PAIR_PER_TILE + p, pl.ds(q * _LANE, _LANE)]
                    return ab + va * vb, aa + va * va, bb + vb * vb

                z = jnp.zeros((_LANE,), jnp.float32)
                ab, aa, bb = lax.fori_loop(0, DIM // _LANE, dot_body, (z, z, z))
                dot = _lane_sum(ab)
                na = _lane_sum(aa)
                nb = _lane_sum(bb)
                norm_a = jnp.maximum(na * _rsqrt_vec(na), jnp.float32(1e-12))
                norm_b = jnp.maximum(nb * _rsqrt_vec(nb), jnp.float32(1e-12))
                s = dot / (norm_a * norm_b)
                s = jnp.clip(s, jnp.float32(0.0), jnp.float32(1.0))
                term = jnp.float32(4.0) * s * (jnp.float32(1.0) - s)
                gi = base + p
                ploss = ploss + jnp.where(gi < limit, term,
                                          jnp.zeros((_LANE,), jnp.float32))

            scal_v[...] = jnp.where(
                lax.iota(jnp.int32, _LANE) == 0, ploss, jnp.float32(0.0))
            pltpu.sync_copy(scal_v, parts_hbm.at[sid])

        @pl.when(jnp.logical_and(on0, sid == 0))
        def _():
            lane = lax.iota(jnp.int32, _LANE)
            meta_v[pl.ds(0, _LANE)] = jnp.where(
                lane == 0, cnt, jnp.where(lane == 1, limit, 0))
            pltpu.sync_copy(meta_v, meta_hbm)

    return k(emb, mask_i32, o1a, o2a, o1b, o2b)


def _tc_finish(parts, meta):
    def body(parts_ref, meta_ref, out_ref):
        total = jnp.sum(parts_ref[...])
        cnt = meta_ref[0, 0]
        limit = meta_ref[0, 1]
        denom = jnp.maximum(limit, 1).astype(jnp.float32)
        loss = jnp.float32(WEIGHT) * total / denom
        out_ref[...] = jnp.full((1, 1), jnp.where(cnt >= 2, loss,
                                                  jnp.float32(0.0)))

    return pl.pallas_call(
        body,
        out_shape=jax.ShapeDtypeStruct((1, 1), jnp.float32),
    )(parts, meta)


def kernel(embeddings, unknown_mask):
    mask_i32 = unknown_mask.astype(jnp.int32)
    parts, meta = _sc_call(
        embeddings,
        mask_i32,
        jnp.asarray(_ORD1A),
        jnp.asarray(_ORD2A),
        jnp.asarray(_ORD1B),
        jnp.asarray(_ORD2B),
    )
    out = _tc_finish(parts, meta.reshape(1, _LANE))
    return out[0, 0]


# speedup vs baseline: 1.5767x; 1.0857x over previous
import functools

import numpy as np
import jax
import jax.numpy as jnp
from jax import lax
from jax.experimental import pallas as pl
from jax.experimental.pallas import tpu as pltpu
from jax.experimental.pallas import tpu_sc as plsc

TOTAL = 16384
DIM = 1024
NSAMP = 512
SMALL_CNT = 1625
WEIGHT = 1.0


def _tf2x32(k0, k1, x0, x1):
    k0, k1 = np.uint32(k0), np.uint32(k1)
    x0, x1 = np.asarray(x0, np.uint32), np.asarray(x1, np.uint32)
    rotations = ((13, 15, 26, 6), (17, 29, 16, 24))
    ks = (k0, k1, np.uint32(k0 ^ k1 ^ np.uint32(0x1BD11BDA)))
    x0 = x0 + ks[0]
    x1 = x1 + ks[1]
    for i in range(5):
        for r in rotations[i % 2]:
            x0 = x0 + x1
            x1 = ((x1 << np.uint32(r)) | (x1 >> np.uint32(32 - r))) ^ x0
        x0 = x0 + ks[(i + 1) % 3]
        x1 = x1 + ks[(i + 2) % 3] + np.uint32(i + 1)
    return x0, x1


def _np_split(k):
    b1, b2 = _tf2x32(k[0], k[1], np.zeros(2, np.uint32),
                     np.arange(2, dtype=np.uint32))
    return (b1[0], b2[0]), (b1[1], b2[1])


def _np_bits(k, n):
    b1, b2 = _tf2x32(k[0], k[1], np.zeros(n, np.uint32),
                     np.arange(n, dtype=np.uint32))
    return b1 ^ b2


def _shuffle_orders():
    old = np.seterr(over="ignore")
    try:
        ka, kb = _np_split((np.uint32(0), np.uint32(1234)))
        orders = []
        for base in (ka, kb):
            knext, sub1 = _np_split(base)
            _, sub2 = _np_split(knext)
            for sub in (sub1, sub2):
                bits = _np_bits(sub, TOTAL)
                orders.append(np.argsort(bits, kind="stable").astype(np.int32))
        return orders
    finally:
        np.seterr(**old)


_ORD1A, _ORD2A, _ORD1B, _ORD2B = _shuffle_orders()


_LANE = 16
_NCHUNK = TOTAL // _LANE
_PAIR_PER_TILE = NSAMP // 16


def _lane_sum(v):
    for s in range(4):
        idx = lax.iota(jnp.int32, _LANE) ^ (1 << s)
        v = v + jnp.take_along_axis(v, idx, axis=0, mode="promise_in_bounds")
    return v


def _rsqrt_vec(x):
    i = lax.bitcast_convert_type(x, jnp.int32)
    i = jnp.int32(0x5F3759DF) - lax.shift_right_arithmetic(i, 1)
    y = lax.bitcast_convert_type(i, jnp.float32)
    for _ in range(3):
        y = y * (jnp.float32(1.5) - jnp.float32(0.5) * x * y * y)
    return y


def _sc_call(emb, mask_i32, o1a, o2a, o1b, o2b):
    mesh = plsc.VectorSubcoreMesh(core_axis_name="c", subcore_axis_name="s")

    @functools.partial(
        pl.kernel,
        out_type=[
            jax.ShapeDtypeStruct((_LANE, _LANE), jnp.float32),
            jax.ShapeDtypeStruct((_LANE,), jnp.int32),
        ],
        mesh=mesh,
        compiler_params=pltpu.CompilerParams(needs_layout_passes=False),
        scratch_types=[
            pltpu.VMEM((TOTAL,), jnp.int32),
            pltpu.VMEM((TOTAL,), jnp.int32),
            pltpu.VMEM((TOTAL + _LANE,), jnp.int32),
            pltpu.VMEM((NSAMP,), jnp.int32),
            pltpu.VMEM((NSAMP,), jnp.int32),
            pltpu.VMEM((NSAMP,), jnp.int32),
            pltpu.VMEM((NSAMP,), jnp.int32),
            pltpu.VMEM((2 * _PAIR_PER_TILE,), jnp.int32),
            pltpu.VMEM((2 * _PAIR_PER_TILE, DIM), jnp.float32),
            pltpu.VMEM((_LANE,), jnp.float32),
            pltpu.VMEM((_LANE,), jnp.int32),
            pltpu.VMEM_SHARED((2, NSAMP), jnp.int32),
            pltpu.VMEM_SHARED((2, NSAMP), jnp.int32),
            pltpu.VMEM_SHARED((2, NSAMP), jnp.int32),
            pltpu.SemaphoreType.DMA,
        ],
    )
    def k(emb_hbm, mask_hbm, o1a_hbm, o2a_hbm, o1b_hbm, o2b_hbm,
          parts_hbm, meta_hbm,
          mask_v, in_v, comp_v, jl_v, ix_v, ix2_v, sel_v, idx_v,
          rows_v, scal_v, meta_v,
          sh_j, sh_idx, sh_sel, sema):
        cid = lax.axis_index("c")
        sid = lax.axis_index("s")
        on0 = cid == 0

        pltpu.sync_copy(mask_hbm, mask_v)
        pl.delay(500)

        def cnt_body(kk, accs):
            b = kk * (4 * _LANE)
            return tuple(
                a + mask_v[pl.ds(b + j * _LANE, _LANE)]
                for j, a in enumerate(accs))

        z4 = (jnp.zeros((_LANE,), jnp.int32),) * 4
        accs = lax.fori_loop(0, _NCHUNK // 4, cnt_body, z4)
        cnt = _lane_sum(accs[0] + accs[1] + accs[2] + accs[3])[0]
        limit = jnp.minimum(cnt, jnp.int32(NSAMP))
        two_round = cnt > jnp.int32(SMALL_CNT)

        def compact_step(v, m, off):
            key = (lax.iota(jnp.int32, _LANE)
                   + jnp.where(m, 0, _LANE)).astype(jnp.uint32)
            _, sv = plsc.sort_key_val(key, v)
            comp_v[pl.ds(off, _LANE)] = sv
            pc = plsc.all_reduce_population_count(m)
            return off + pc[0]

        def compact_ord(src_hbm):
            pltpu.sync_copy(src_hbm, in_v)
            pl.delay(500)

            def body(kk, off):
                b = kk * (4 * _LANE)
                for j in range(4):
                    v = in_v[pl.ds(b + j * _LANE, _LANE)]
                    off = compact_step(v, v < cnt, off)
                return off

            lax.fori_loop(0, _NCHUNK // 4, body, jnp.int32(0))

        @pl.when(jnp.logical_and(on0, sid == 1))
        def _():
            compact_ord(o1a_hbm)

        @pl.when(jnp.logical_and(on0, sid == 2))
        def _():
            compact_ord(o1b_hbm)

        @pl.when(jnp.logical_and(on0, sid == 3))
        def _():
            compact_ord(o2a_hbm)
            pltpu.sync_copy(comp_v.at[pl.ds(0, NSAMP)], sh_j.at[0])

        @pl.when(jnp.logical_and(on0, sid == 4))
        def _():
            compact_ord(o2b_hbm)
            pltpu.sync_copy(comp_v.at[pl.ds(0, NSAMP)], sh_j.at[1])

        @pl.when(jnp.logical_and(on0, sid == 5))
        def _():
            def body(kk, off):
                b = kk * (4 * _LANE)
                for j in range(4):
                    mv = mask_v[pl.ds(b + j * _LANE, _LANE)]
                    ids = b + j * _LANE + lax.iota(jnp.int32, _LANE)
                    off = compact_step(ids, mv > 0, off)
                return off

            lax.fori_loop(0, _NCHUNK // 4, body, jnp.int32(0))

        plsc.subcore_barrier()

        def compose(row):
            pltpu.sync_copy(sh_j.at[row], jl_v)
            pl.delay(1500)

            def body(q, _):
                ivec = q * _LANE + lax.iota(jnp.int32, _LANE)
                jv = jnp.where(two_round,
                               jl_v[pl.ds(q * _LANE, _LANE)], ivec)
                av = plsc.load_gather(comp_v, [jv])
                av = jnp.where(ivec < limit, av, 0)
                ix_v[pl.ds(q * _LANE, _LANE)] = av
                return 0

            lax.fori_loop(0, NSAMP // _LANE, body, 0)
            pltpu.sync_copy(ix_v, sh_idx.at[row])

        @pl.when(jnp.logical_and(on0, sid == 1))
        def _():
            compose(0)

        @pl.when(jnp.logical_and(on0, sid == 2))
        def _():
            compose(1)

        plsc.subcore_barrier()

        @pl.when(jnp.logical_and(on0, sid == 5))
        def _():
            pltpu.sync_copy(sh_idx.at[0], ix_v)
            pltpu.sync_copy(sh_idx.at[1], ix2_v)
            pl.delay(1500)

            def sel_one(src_v, row):
                def body(q, _):
                    ivec = q * _LANE + lax.iota(jnp.int32, _LANE)
                    xs = src_v[pl.ds(q * _LANE, _LANE)]
                    sv = plsc.load_gather(comp_v, [xs])
                    sv = jnp.where(ivec < limit, sv, 0)
                    sel_v[pl.ds(q * _LANE, _LANE)] = sv
                    return 0

                lax.fori_loop(0, NSAMP // _LANE, body, 0)
                pltpu.sync_copy(sel_v, sh_sel.at[row])

            sel_one(ix_v, 0)
            sel_one(ix2_v, 1)

        plsc.subcore_barrier()

        @pl.when(on0)
        def _():
            base = sid * _PAIR_PER_TILE
            pltpu.sync_copy(sh_sel.at[0, pl.ds(base, _PAIR_PER_TILE)],
                            idx_v.at[pl.ds(0, _PAIR_PER_TILE)])
            pltpu.sync_copy(sh_sel.at[1, pl.ds(base, _PAIR_PER_TILE)],
                            idx_v.at[pl.ds(_PAIR_PER_TILE, _PAIR_PER_TILE)])
            pltpu.async_copy(emb_hbm.at[idx_v], rows_v, sema).wait()
            pl.delay(2500)

            ploss = jnp.zeros((_LANE,), jnp.float32)
            for p in range(_PAIR_PER_TILE):
                def dot_body(q, accs):
                    accs = list(accs)
                    b = q * (4 * _LANE)
                    for j in range(4):
                        va = rows_v[p, pl.ds(b + j * _LANE, _LANE)]
                        vb = rows_v[_PAIR_PER_TILE + p,
                                    pl.ds(b + j * _LANE, _LANE)]
                        t = (j % 2) * 3
                        accs[t] = accs[t] + va * vb
                        accs[t + 1] = accs[t + 1] + va * va
                        accs[t + 2] = accs[t + 2] + vb * vb
                    return tuple(accs)

                z6 = (jnp.zeros((_LANE,), jnp.float32),) * 6
                r6 = lax.fori_loop(0, DIM // (4 * _LANE), dot_body, z6)
                dot = _lane_sum(r6[0] + r6[3])
                na = _lane_sum(r6[1] + r6[4])
                nb = _lane_sum(r6[2] + r6[5])
                norm_a = jnp.maximum(na * _rsqrt_vec(na), jnp.float32(1e-12))
                norm_b = jnp.maximum(nb * _rsqrt_vec(nb), jnp.float32(1e-12))
                s = dot / (norm_a * norm_b)
                s = jnp.clip(s, jnp.float32(0.0), jnp.float32(1.0))
                term = jnp.float32(4.0) * s * (jnp.float32(1.0) - s)
                gi = base + p
                ploss = ploss + jnp.where(gi < limit, term,
                                          jnp.zeros((_LANE,), jnp.float32))

            scal_v[...] = jnp.where(
                lax.iota(jnp.int32, _LANE) == 0, ploss, jnp.float32(0.0))
            pltpu.sync_copy(scal_v, parts_hbm.at[sid])

        @pl.when(jnp.logical_and(on0, sid == 0))
        def _():
            lane = lax.iota(jnp.int32, _LANE)
            meta_v[pl.ds(0, _LANE)] = jnp.where(
                lane == 0, cnt, jnp.where(lane == 1, limit, 0))
            pltpu.sync_copy(meta_v, meta_hbm)

    return k(emb, mask_i32, o1a, o2a, o1b, o2b)


def _tc_finish(parts, meta):
    def body(parts_ref, meta_ref, out_ref):
        total = jnp.sum(parts_ref[...])
        cnt = meta_ref[0, 0]
        limit = meta_ref[0, 1]
        denom = jnp.maximum(limit, 1).astype(jnp.float32)
        loss = jnp.float32(WEIGHT) * total / denom
        out_ref[...] = jnp.full((1, 1), jnp.where(cnt >= 2, loss,
                                                  jnp.float32(0.0)))

    return pl.pallas_call(
        body,
        out_shape=jax.ShapeDtypeStruct((1, 1), jnp.float32),
    )(parts, meta)


def kernel(embeddings, unknown_mask):
    mask_i32 = unknown_mask.astype(jnp.int32)
    parts, meta = _sc_call(
        embeddings,
        mask_i32,
        jnp.asarray(_ORD1A),
        jnp.asarray(_ORD2A),
        jnp.asarray(_ORD1B),
        jnp.asarray(_ORD2B),
    )
    out = _tc_finish(parts, meta.reshape(1, _LANE))
    return out[0, 0]


# speedup vs baseline: 1.6922x; 1.0732x over previous
import functools

import numpy as np
import jax
import jax.numpy as jnp
from jax import lax
from jax.experimental import pallas as pl
from jax.experimental.pallas import tpu as pltpu
from jax.experimental.pallas import tpu_sc as plsc

TOTAL = 16384
DIM = 1024
NSAMP = 512
SMALL_CNT = 1625
WEIGHT = 1.0


def _tf2x32(k0, k1, x0, x1):
    k0, k1 = np.uint32(k0), np.uint32(k1)
    x0, x1 = np.asarray(x0, np.uint32), np.asarray(x1, np.uint32)
    rotations = ((13, 15, 26, 6), (17, 29, 16, 24))
    ks = (k0, k1, np.uint32(k0 ^ k1 ^ np.uint32(0x1BD11BDA)))
    x0 = x0 + ks[0]
    x1 = x1 + ks[1]
    for i in range(5):
        for r in rotations[i % 2]:
            x0 = x0 + x1
            x1 = ((x1 << np.uint32(r)) | (x1 >> np.uint32(32 - r))) ^ x0
        x0 = x0 + ks[(i + 1) % 3]
        x1 = x1 + ks[(i + 2) % 3] + np.uint32(i + 1)
    return x0, x1


def _np_split(k):
    b1, b2 = _tf2x32(k[0], k[1], np.zeros(2, np.uint32),
                     np.arange(2, dtype=np.uint32))
    return (b1[0], b2[0]), (b1[1], b2[1])


def _np_bits(k, n):
    b1, b2 = _tf2x32(k[0], k[1], np.zeros(n, np.uint32),
                     np.arange(n, dtype=np.uint32))
    return b1 ^ b2


def _shuffle_orders():
    old = np.seterr(over="ignore")
    try:
        ka, kb = _np_split((np.uint32(0), np.uint32(1234)))
        orders = []
        for base in (ka, kb):
            knext, sub1 = _np_split(base)
            _, sub2 = _np_split(knext)
            for sub in (sub1, sub2):
                bits = _np_bits(sub, TOTAL)
                orders.append(np.argsort(bits, kind="stable").astype(np.int32))
        return orders
    finally:
        np.seterr(**old)


_ORD1A, _ORD2A, _ORD1B, _ORD2B = _shuffle_orders()


_LANE = 16
_NCHUNK = TOTAL // _LANE
_PAIR_PER_TILE = NSAMP // 32


def _lane_sum(v):
    for s in range(4):
        idx = lax.iota(jnp.int32, _LANE) ^ (1 << s)
        v = v + jnp.take_along_axis(v, idx, axis=0, mode="promise_in_bounds")
    return v


def _rsqrt_vec(x):
    i = lax.bitcast_convert_type(x, jnp.int32)
    i = jnp.int32(0x5F3759DF) - lax.shift_right_arithmetic(i, 1)
    y = lax.bitcast_convert_type(i, jnp.float32)
    for _ in range(3):
        y = y * (jnp.float32(1.5) - jnp.float32(0.5) * x * y * y)
    return y


def _sc_call(emb, mask_i32, o1a, o2a, o1b, o2b):
    mesh = plsc.VectorSubcoreMesh(core_axis_name="c", subcore_axis_name="s")

    @functools.partial(
        pl.kernel,
        out_type=[
            jax.ShapeDtypeStruct((2 * _LANE, _LANE), jnp.float32),
            jax.ShapeDtypeStruct((_LANE,), jnp.int32),
        ],
        mesh=mesh,
        compiler_params=pltpu.CompilerParams(needs_layout_passes=False),
        scratch_types=[
            pltpu.VMEM((TOTAL,), jnp.int32),
            pltpu.VMEM((TOTAL,), jnp.int32),
            pltpu.VMEM((TOTAL + _LANE,), jnp.int32),
            pltpu.VMEM((NSAMP,), jnp.int32),
            pltpu.VMEM((NSAMP,), jnp.int32),
            pltpu.VMEM((NSAMP,), jnp.int32),
            pltpu.VMEM((NSAMP,), jnp.int32),
            pltpu.VMEM((2 * _PAIR_PER_TILE,), jnp.int32),
            pltpu.VMEM((2 * _PAIR_PER_TILE, DIM), jnp.float32),
            pltpu.VMEM((_LANE,), jnp.float32),
            pltpu.VMEM((_LANE,), jnp.int32),
            pltpu.VMEM_SHARED((2, NSAMP), jnp.int32),
            pltpu.VMEM_SHARED((2, NSAMP), jnp.int32),
            pltpu.VMEM_SHARED((2, NSAMP), jnp.int32),
            pltpu.SemaphoreType.DMA,
        ],
    )
    def k(emb_hbm, mask_hbm, o1a_hbm, o2a_hbm, o1b_hbm, o2b_hbm,
          parts_hbm, meta_hbm,
          mask_v, in_v, comp_v, jl_v, ix_v, ix2_v, sel_v, idx_v,
          rows_v, scal_v, meta_v,
          sh_j, sh_idx, sh_sel, sema):
        cid = lax.axis_index("c")
        sid = lax.axis_index("s")
        on0 = cid == 0

        pltpu.sync_copy(mask_hbm, mask_v)
        pl.delay(500)

        def cnt_body(kk, accs):
            b = kk * (4 * _LANE)
            return tuple(
                a + mask_v[pl.ds(b + j * _LANE, _LANE)]
                for j, a in enumerate(accs))

        z4 = (jnp.zeros((_LANE,), jnp.int32),) * 4
        accs = lax.fori_loop(0, _NCHUNK // 4, cnt_body, z4)
        cnt = _lane_sum(accs[0] + accs[1] + accs[2] + accs[3])[0]
        limit = jnp.minimum(cnt, jnp.int32(NSAMP))
        two_round = cnt > jnp.int32(SMALL_CNT)

        def compact_step(v, m, off):
            key = (lax.iota(jnp.int32, _LANE)
                   + jnp.where(m, 0, _LANE)).astype(jnp.uint32)
            _, sv = plsc.sort_key_val(key, v)
            comp_v[pl.ds(off, _LANE)] = sv
            pc = plsc.all_reduce_population_count(m)
            return off + pc[0]

        def compact_ord(src_hbm):
            pltpu.sync_copy(src_hbm, in_v)
            pl.delay(500)

            def body(kk, off):
                b = kk * (4 * _LANE)
                for j in range(4):
                    v = in_v[pl.ds(b + j * _LANE, _LANE)]
                    off = compact_step(v, v < cnt, off)
                return off

            lax.fori_loop(0, _NCHUNK // 4, body, jnp.int32(0))

        @pl.when(sid == 1)
        def _():
            compact_ord(o1a_hbm)

        @pl.when(sid == 2)
        def _():
            compact_ord(o1b_hbm)

        @pl.when(sid == 3)
        def _():
            compact_ord(o2a_hbm)
            pltpu.sync_copy(comp_v.at[pl.ds(0, NSAMP)], sh_j.at[0])

        @pl.when(sid == 4)
        def _():
            compact_ord(o2b_hbm)
            pltpu.sync_copy(comp_v.at[pl.ds(0, NSAMP)], sh_j.at[1])

        @pl.when(sid == 5)
        def _():
            def body(kk, off):
                b = kk * (4 * _LANE)
                for j in range(4):
                    mv = mask_v[pl.ds(b + j * _LANE, _LANE)]
                    ids = b + j * _LANE + lax.iota(jnp.int32, _LANE)
                    off = compact_step(ids, mv > 0, off)
                return off

            lax.fori_loop(0, _NCHUNK // 4, body, jnp.int32(0))

        plsc.subcore_barrier()

        def compose(row):
            pltpu.sync_copy(sh_j.at[row], jl_v)
            pl.delay(1500)

            def body(q, _):
                ivec = q * _LANE + lax.iota(jnp.int32, _LANE)
                jv = jnp.where(two_round,
                               jl_v[pl.ds(q * _LANE, _LANE)], ivec)
                av = plsc.load_gather(comp_v, [jv])
                av = jnp.where(ivec < limit, av, 0)
                ix_v[pl.ds(q * _LANE, _LANE)] = av
                return 0

            lax.fori_loop(0, NSAMP // _LANE, body, 0)
            pltpu.sync_copy(ix_v, sh_idx.at[row])

        @pl.when(sid == 1)
        def _():
            compose(0)

        @pl.when(sid == 2)
        def _():
            compose(1)

        plsc.subcore_barrier()

        @pl.when(sid == 5)
        def _():
            pltpu.sync_copy(sh_idx.at[0], ix_v)
            pltpu.sync_copy(sh_idx.at[1], ix2_v)
            pl.delay(1500)

            def sel_one(src_v, row):
                def body(q, _):
                    ivec = q * _LANE + lax.iota(jnp.int32, _LANE)
                    xs = src_v[pl.ds(q * _LANE, _LANE)]
                    sv = plsc.load_gather(comp_v, [xs])
                    sv = jnp.where(ivec < limit, sv, 0)
                    sel_v[pl.ds(q * _LANE, _LANE)] = sv
                    return 0

                lax.fori_loop(0, NSAMP // _LANE, body, 0)
                pltpu.sync_copy(sel_v, sh_sel.at[row])

            sel_one(ix_v, 0)
            sel_one(ix2_v, 1)

        plsc.subcore_barrier()

        wid = cid * _LANE + sid
        if True:
            base = wid * _PAIR_PER_TILE
            pltpu.sync_copy(sh_sel.at[0, pl.ds(base, _PAIR_PER_TILE)],
                            idx_v.at[pl.ds(0, _PAIR_PER_TILE)])
            pltpu.sync_copy(sh_sel.at[1, pl.ds(base, _PAIR_PER_TILE)],
                            idx_v.at[pl.ds(_PAIR_PER_TILE, _PAIR_PER_TILE)])
            pltpu.async_copy(emb_hbm.at[idx_v], rows_v, sema).wait()
            pl.delay(2500)

            ploss = jnp.zeros((_LANE,), jnp.float32)
            for p in range(_PAIR_PER_TILE):
                def dot_body(q, accs):
                    accs = list(accs)
                    b = q * (4 * _LANE)
                    for j in range(4):
                        va = rows_v[p, pl.ds(b + j * _LANE, _LANE)]
                        vb = rows_v[_PAIR_PER_TILE + p,
                                    pl.ds(b + j * _LANE, _LANE)]
                        t = (j % 2) * 3
                        accs[t] = accs[t] + va * vb
                        accs[t + 1] = accs[t + 1] + va * va
                        accs[t + 2] = accs[t + 2] + vb * vb
                    return tuple(accs)

                z6 = (jnp.zeros((_LANE,), jnp.float32),) * 6
                r6 = lax.fori_loop(0, DIM // (4 * _LANE), dot_body, z6)
                dot = _lane_sum(r6[0] + r6[3])
                na = _lane_sum(r6[1] + r6[4])
                nb = _lane_sum(r6[2] + r6[5])
                norm_a = jnp.maximum(na * _rsqrt_vec(na), jnp.float32(1e-12))
                norm_b = jnp.maximum(nb * _rsqrt_vec(nb), jnp.float32(1e-12))
                s = dot / (norm_a * norm_b)
                s = jnp.clip(s, jnp.float32(0.0), jnp.float32(1.0))
                term = jnp.float32(4.0) * s * (jnp.float32(1.0) - s)
                gi = base + p
                ploss = ploss + jnp.where(gi < limit, term,
                                          jnp.zeros((_LANE,), jnp.float32))

            scal_v[...] = jnp.where(
                lax.iota(jnp.int32, _LANE) == 0, ploss, jnp.float32(0.0))
            pltpu.sync_copy(scal_v, parts_hbm.at[wid])

        @pl.when(jnp.logical_and(on0, sid == 0))
        def _():
            lane = lax.iota(jnp.int32, _LANE)
            meta_v[pl.ds(0, _LANE)] = jnp.where(
                lane == 0, cnt, jnp.where(lane == 1, limit, 0))
            pltpu.sync_copy(meta_v, meta_hbm)

    return k(emb, mask_i32, o1a, o2a, o1b, o2b)


def _tc_finish(parts, meta):
    def body(parts_ref, meta_ref, out_ref):
        total = jnp.sum(parts_ref[...])
        cnt = meta_ref[0, 0]
        limit = meta_ref[0, 1]
        denom = jnp.maximum(limit, 1).astype(jnp.float32)
        loss = jnp.float32(WEIGHT) * total / denom
        out_ref[...] = jnp.full((1, 1), jnp.where(cnt >= 2, loss,
                                                  jnp.float32(0.0)))

    return pl.pallas_call(
        body,
        out_shape=jax.ShapeDtypeStruct((1, 1), jnp.float32),
    )(parts, meta)


def kernel(embeddings, unknown_mask):
    mask_i32 = unknown_mask.astype(jnp.int32)
    parts, meta = _sc_call(
        embeddings,
        mask_i32,
        jnp.asarray(_ORD1A),
        jnp.asarray(_ORD2A),
        jnp.asarray(_ORD1B),
        jnp.asarray(_ORD2B),
    )
    out = _tc_finish(parts, meta.reshape(1, _LANE))
    return out[0, 0]


# speedup vs baseline: 1.7430x; 1.0300x over previous
import functools

import numpy as np
import jax
import jax.numpy as jnp
from jax import lax
from jax.experimental import pallas as pl
from jax.experimental.pallas import tpu as pltpu
from jax.experimental.pallas import tpu_sc as plsc

TOTAL = 16384
DIM = 1024
NSAMP = 512
SMALL_CNT = 1625
WEIGHT = 1.0


def _tf2x32(k0, k1, x0, x1):
    k0, k1 = np.uint32(k0), np.uint32(k1)
    x0, x1 = np.asarray(x0, np.uint32), np.asarray(x1, np.uint32)
    rotations = ((13, 15, 26, 6), (17, 29, 16, 24))
    ks = (k0, k1, np.uint32(k0 ^ k1 ^ np.uint32(0x1BD11BDA)))
    x0 = x0 + ks[0]
    x1 = x1 + ks[1]
    for i in range(5):
        for r in rotations[i % 2]:
            x0 = x0 + x1
            x1 = ((x1 << np.uint32(r)) | (x1 >> np.uint32(32 - r))) ^ x0
        x0 = x0 + ks[(i + 1) % 3]
        x1 = x1 + ks[(i + 2) % 3] + np.uint32(i + 1)
    return x0, x1


def _np_split(k):
    b1, b2 = _tf2x32(k[0], k[1], np.zeros(2, np.uint32),
                     np.arange(2, dtype=np.uint32))
    return (b1[0], b2[0]), (b1[1], b2[1])


def _np_bits(k, n):
    b1, b2 = _tf2x32(k[0], k[1], np.zeros(n, np.uint32),
                     np.arange(n, dtype=np.uint32))
    return b1 ^ b2


def _shuffle_orders():
    old = np.seterr(over="ignore")
    try:
        ka, kb = _np_split((np.uint32(0), np.uint32(1234)))
        orders = []
        for base in (ka, kb):
            knext, sub1 = _np_split(base)
            _, sub2 = _np_split(knext)
            for sub in (sub1, sub2):
                bits = _np_bits(sub, TOTAL)
                orders.append(np.argsort(bits, kind="stable").astype(np.int32))
        return orders
    finally:
        np.seterr(**old)


_ORD1A, _ORD2A, _ORD1B, _ORD2B = _shuffle_orders()


_LANE = 16
_NCHUNK = TOTAL // _LANE
_PAIR_PER_TILE = NSAMP // 32


def _lane_sum(v):
    for s in range(4):
        idx = lax.iota(jnp.int32, _LANE) ^ (1 << s)
        v = v + jnp.take_along_axis(v, idx, axis=0, mode="promise_in_bounds")
    return v


def _rsqrt_vec(x):
    i = lax.bitcast_convert_type(x, jnp.int32)
    i = jnp.int32(0x5F3759DF) - lax.shift_right_arithmetic(i, 1)
    y = lax.bitcast_convert_type(i, jnp.float32)
    for _ in range(3):
        y = y * (jnp.float32(1.5) - jnp.float32(0.5) * x * y * y)
    return y


def _sc_call(emb, mask_i32, o1a, o2a, o1b, o2b):
    mesh = plsc.VectorSubcoreMesh(core_axis_name="c", subcore_axis_name="s")

    @functools.partial(
        pl.kernel,
        out_type=[
            jax.ShapeDtypeStruct((2 * _LANE, _LANE), jnp.float32),
            jax.ShapeDtypeStruct((_LANE,), jnp.int32),
        ],
        mesh=mesh,
        compiler_params=pltpu.CompilerParams(needs_layout_passes=False),
        scratch_types=[
            pltpu.VMEM((TOTAL,), jnp.int32),
            pltpu.VMEM((TOTAL,), jnp.int32),
            pltpu.VMEM((TOTAL + _LANE,), jnp.int32),
            pltpu.VMEM((NSAMP,), jnp.int32),
            pltpu.VMEM((NSAMP,), jnp.int32),
            pltpu.VMEM((NSAMP,), jnp.int32),
            pltpu.VMEM((NSAMP,), jnp.int32),
            pltpu.VMEM((2 * _PAIR_PER_TILE,), jnp.int32),
            pltpu.VMEM((2 * _PAIR_PER_TILE, DIM), jnp.float32),
            pltpu.VMEM((_LANE,), jnp.float32),
            pltpu.VMEM((_LANE,), jnp.int32),
            pltpu.VMEM_SHARED((2, NSAMP), jnp.int32),
            pltpu.VMEM_SHARED((2, NSAMP), jnp.int32),
            pltpu.VMEM_SHARED((2, NSAMP), jnp.int32),
            pltpu.SemaphoreType.DMA,
        ],
    )
    def k(emb_hbm, mask_hbm, o1a_hbm, o2a_hbm, o1b_hbm, o2b_hbm,
          parts_hbm, meta_hbm,
          mask_v, in_v, comp_v, jl_v, ix_v, ix2_v, sel_v, idx_v,
          rows_v, scal_v, meta_v,
          sh_j, sh_idx, sh_sel, sema):
        cid = lax.axis_index("c")
        sid = lax.axis_index("s")
        on0 = cid == 0

        pltpu.sync_copy(mask_hbm, mask_v)
        pl.delay(500)

        def cnt_body(kk, accs):
            b = kk * (4 * _LANE)
            return tuple(
                a + mask_v[pl.ds(b + j * _LANE, _LANE)]
                for j, a in enumerate(accs))

        z4 = (jnp.zeros((_LANE,), jnp.int32),) * 4
        accs = lax.fori_loop(0, _NCHUNK // 4, cnt_body, z4)
        cnt = _lane_sum(accs[0] + accs[1] + accs[2] + accs[3])[0]
        limit = jnp.minimum(cnt, jnp.int32(NSAMP))
        two_round = cnt > jnp.int32(SMALL_CNT)

        def compact_step(v, m, off):
            key = (lax.iota(jnp.int32, _LANE)
                   + jnp.where(m, 0, _LANE)).astype(jnp.uint32)
            _, sv = plsc.sort_key_val(key, v)
            comp_v[pl.ds(off, _LANE)] = sv
            pc = plsc.all_reduce_population_count(m)
            return off + pc[0]

        def compact_ord(src_hbm):
            pltpu.sync_copy(src_hbm, in_v)
            pl.delay(500)

            def body(kk, off):
                b = kk * (4 * _LANE)
                for j in range(4):
                    v = in_v[pl.ds(b + j * _LANE, _LANE)]
                    off = compact_step(v, v < cnt, off)
                return off

            lax.fori_loop(0, _NCHUNK // 4, body, jnp.int32(0))

        @pl.when(sid == 1)
        def _():
            compact_ord(o1a_hbm)

        @pl.when(sid == 2)
        def _():
            compact_ord(o1b_hbm)

        @pl.when(sid == 3)
        def _():
            compact_ord(o2a_hbm)
            pltpu.sync_copy(comp_v.at[pl.ds(0, NSAMP)], sh_j.at[0])

        @pl.when(sid == 4)
        def _():
            compact_ord(o2b_hbm)
            pltpu.sync_copy(comp_v.at[pl.ds(0, NSAMP)], sh_j.at[1])

        @pl.when(sid == 5)
        def _():
            def body(kk, off):
                b = kk * (4 * _LANE)
                for j in range(4):
                    mv = mask_v[pl.ds(b + j * _LANE, _LANE)]
                    ids = b + j * _LANE + lax.iota(jnp.int32, _LANE)
                    off = compact_step(ids, mv > 0, off)
                return off

            lax.fori_loop(0, _NCHUNK // 4, body, jnp.int32(0))

        plsc.subcore_barrier()

        def compose(row):
            pltpu.sync_copy(sh_j.at[row], jl_v)
            pl.delay(600)

            def body(q, _):
                ivec = q * _LANE + lax.iota(jnp.int32, _LANE)
                jv = jnp.where(two_round,
                               jl_v[pl.ds(q * _LANE, _LANE)], ivec)
                av = plsc.load_gather(comp_v, [jv])
                av = jnp.where(ivec < limit, av, 0)
                ix_v[pl.ds(q * _LANE, _LANE)] = av
                return 0

            lax.fori_loop(0, NSAMP // _LANE, body, 0)
            pltpu.sync_copy(ix_v, sh_idx.at[row])

        @pl.when(sid == 1)
        def _():
            compose(0)

        @pl.when(sid == 2)
        def _():
            compose(1)

        plsc.subcore_barrier()

        @pl.when(sid == 5)
        def _():
            pltpu.sync_copy(sh_idx.at[0], ix_v)
            pltpu.sync_copy(sh_idx.at[1], ix2_v)
            pl.delay(600)

            def sel_one(src_v, row):
                def body(q, _):
                    ivec = q * _LANE + lax.iota(jnp.int32, _LANE)
                    xs = src_v[pl.ds(q * _LANE, _LANE)]
                    sv = plsc.load_gather(comp_v, [xs])
                    sv = jnp.where(ivec < limit, sv, 0)
                    sel_v[pl.ds(q * _LANE, _LANE)] = sv
                    return 0

                lax.fori_loop(0, NSAMP // _LANE, body, 0)
                pltpu.sync_copy(sel_v, sh_sel.at[row])

            sel_one(ix_v, 0)
            sel_one(ix2_v, 1)

        plsc.subcore_barrier()

        wid = cid * _LANE + sid
        if True:
            base = wid * _PAIR_PER_TILE
            pltpu.sync_copy(sh_sel.at[0, pl.ds(base, _PAIR_PER_TILE)],
                            idx_v.at[pl.ds(0, _PAIR_PER_TILE)])
            pltpu.sync_copy(sh_sel.at[1, pl.ds(base, _PAIR_PER_TILE)],
                            idx_v.at[pl.ds(_PAIR_PER_TILE, _PAIR_PER_TILE)])
            pltpu.async_copy(emb_hbm.at[idx_v], rows_v, sema).wait()
            pl.delay(2500)

            ploss = jnp.zeros((_LANE,), jnp.float32)
            for p in range(_PAIR_PER_TILE):
                def dot_body(q, accs):
                    accs = list(accs)
                    b = q * (4 * _LANE)
                    for j in range(4):
                        va = rows_v[p, pl.ds(b + j * _LANE, _LANE)]
                        vb = rows_v[_PAIR_PER_TILE + p,
                                    pl.ds(b + j * _LANE, _LANE)]
                        t = (j % 2) * 3
                        accs[t] = accs[t] + va * vb
                        accs[t + 1] = accs[t + 1] + va * va
                        accs[t + 2] = accs[t + 2] + vb * vb
                    return tuple(accs)

                z6 = (jnp.zeros((_LANE,), jnp.float32),) * 6
                r6 = lax.fori_loop(0, DIM // (4 * _LANE), dot_body, z6)
                dot = _lane_sum(r6[0] + r6[3])
                na = _lane_sum(r6[1] + r6[4])
                nb = _lane_sum(r6[2] + r6[5])
                norm_a = jnp.maximum(na * _rsqrt_vec(na), jnp.float32(1e-12))
                norm_b = jnp.maximum(nb * _rsqrt_vec(nb), jnp.float32(1e-12))
                s = dot / (norm_a * norm_b)
                s = jnp.clip(s, jnp.float32(0.0), jnp.float32(1.0))
                term = jnp.float32(4.0) * s * (jnp.float32(1.0) - s)
                gi = base + p
                ploss = ploss + jnp.where(gi < limit, term,
                                          jnp.zeros((_LANE,), jnp.float32))

            scal_v[...] = jnp.where(
                lax.iota(jnp.int32, _LANE) == 0, ploss, jnp.float32(0.0))
            pltpu.sync_copy(scal_v, parts_hbm.at[wid])

        @pl.when(jnp.logical_and(on0, sid == 0))
        def _():
            lane = lax.iota(jnp.int32, _LANE)
            meta_v[pl.ds(0, _LANE)] = jnp.where(
                lane == 0, cnt, jnp.where(lane == 1, limit, 0))
            pltpu.sync_copy(meta_v, meta_hbm)

    return k(emb, mask_i32, o1a, o2a, o1b, o2b)


def _tc_finish(parts, meta):
    def body(parts_ref, meta_ref, out_ref):
        total = jnp.sum(parts_ref[...])
        cnt = meta_ref[0, 0]
        limit = meta_ref[0, 1]
        denom = jnp.maximum(limit, 1).astype(jnp.float32)
        loss = jnp.float32(WEIGHT) * total / denom
        out_ref[...] = jnp.full((1, 1), jnp.where(cnt >= 2, loss,
                                                  jnp.float32(0.0)))

    return pl.pallas_call(
        body,
        out_shape=jax.ShapeDtypeStruct((1, 1), jnp.float32),
    )(parts, meta)


def kernel(embeddings, unknown_mask):
    mask_i32 = unknown_mask.astype(jnp.int32)
    parts, meta = _sc_call(
        embeddings,
        mask_i32,
        jnp.asarray(_ORD1A),
        jnp.asarray(_ORD2A),
        jnp.asarray(_ORD1B),
        jnp.asarray(_ORD2B),
    )
    out = _tc_finish(parts, meta.reshape(1, _LANE))
    return out[0, 0]
